# baseline (speedup 1.0000x reference)
import functools

import jax
import jax.numpy as jnp
from jax import lax
from jax.experimental import pallas as pl
from jax.experimental.pallas import tpu as pltpu
from jax.experimental.pallas import tpu_sc as plsc

N = 16 * 1 * 512 * 512
LBITS = 14
B = 1 << LBITS
NW = 32
ROWS_PER_CHUNK = 16
NCH = 256 // ROWS_PER_CHUNK
LANES = 16

_INT_MIN = -2147483648


def _sc_hist_kernel(logits_hbm, labels_hbm, out_hbm,
                    lbuf0, ybuf0, lbuf1, ybuf1, lbuf2, ybuf2, hist,
                    sl0, sy0, sl1, sy1, sl2, sy2):
    nc = 2
    wid = lax.axis_index("s") * nc + lax.axis_index("c")
    img = wid // 2
    row_base = (wid % 2) * 256

    zeros16 = jnp.zeros((LANES,), jnp.float32)
    ones16 = jnp.full((LANES,), 1.0, jnp.float32)
    sh31 = jnp.full((LANES,), 31, jnp.int32)
    sh_binrow = jnp.full((LANES,), 32 - LBITS + 7, jnp.int32)
    sh_bin = jnp.full((LANES,), 32 - LBITS, jnp.int32)
    sh7 = jnp.full((LANES,), 7, jnp.int32)

    @plsc.parallel_loop(0, 4 * B // 128, unroll=8)
    def _(r):
        for j in range(128 // LANES):
            hist[r, pl.ds(j * LANES, LANES)] = zeros16

    def process(lb, yb):
        @plsc.parallel_loop(0, ROWS_PER_CHUNK * (512 // LANES), unroll=8)
        def _(i):
            r = i // (512 // LANES)
            c = (i % (512 // LANES)) * LANES
            if True:
                x = lb[r, pl.ds(c, LANES)]
                y = yb[r, pl.ds(c, LANES)]
                y1 = y ^ 1
                xb = lax.bitcast_convert_type(x, jnp.int32)
                e = 1.0 - lax.bitcast_convert_type(
                    xb ^ lax.shift_left(y1, sh31), jnp.float32)
                relu_e = jnp.maximum(e, 0.0)
                b = lax.bitcast_convert_type(e, jnp.int32)
                keyu = b ^ (lax.shift_right_arithmetic(b, sh31)
                            | jnp.int32(_INT_MIN))
                col = lax.shift_right_logical(keyu, sh_bin) & 127
                row_cnt = (lax.shift_right_logical(keyu, sh_binrow)
                           | lax.shift_left(y1, sh7))
                row_sum = row_cnt | (2 * B // 128)
                plsc.addupdate_scatter(hist, [row_cnt, col], ones16)
                plsc.addupdate_scatter(hist, [row_sum, col], relu_e)

    def start(g, lb, yb, sl, sy):
        r0 = row_base + g * ROWS_PER_CHUNK
        pltpu.async_copy(
            logits_hbm.at[img, 0, pl.ds(r0, ROWS_PER_CHUNK), :], lb, sl)
        pltpu.async_copy(
            labels_hbm.at[img, 0, pl.ds(r0, ROWS_PER_CHUNK), :], yb, sy)

    def wait(lb, yb, sl, sy):
        src_l = logits_hbm.at[img, 0, pl.ds(row_base, ROWS_PER_CHUNK), :]
        src_y = labels_hbm.at[img, 0, pl.ds(row_base, ROWS_PER_CHUNK), :]
        pltpu.make_async_copy(src_l, lb, sl).wait()
        pltpu.make_async_copy(src_y, yb, sy).wait()

    bufs = ((lbuf0, ybuf0, sl0, sy0),
            (lbuf1, ybuf1, sl1, sy1),
            (lbuf2, ybuf2, sl2, sy2))

    start(0, *bufs[0])
    start(1, *bufs[1])

    def ring_body(h, c):
        g = 3 * h
        for k in range(3):
            wait(*bufs[k])
            start(jnp.minimum(g + 2 + k, NCH - 1), *bufs[(k + 2) % 3])
            process(bufs[k][0], bufs[k][1])
        return c

    lax.fori_loop(0, (NCH - 1) // 3, ring_body, 0)
    wait(*bufs[0])
    process(bufs[0][0], bufs[0][1])
    wait(*bufs[1])
    pltpu.sync_copy(hist, out_hbm.at[wid])


@functools.cache
def _sc_hist():
    return pl.kernel(
        _sc_hist_kernel,
        mesh=plsc.VectorSubcoreMesh(core_axis_name="c", subcore_axis_name="s"),
        compiler_params=pltpu.CompilerParams(needs_layout_passes=False),
        out_type=jax.ShapeDtypeStruct((NW, 4 * B // 128, 128), jnp.float32),
        scratch_types=[
            pltpu.VMEM((ROWS_PER_CHUNK, 512), jnp.float32),
            pltpu.VMEM((ROWS_PER_CHUNK, 512), jnp.int32),
            pltpu.VMEM((ROWS_PER_CHUNK, 512), jnp.float32),
            pltpu.VMEM((ROWS_PER_CHUNK, 512), jnp.int32),
            pltpu.VMEM((ROWS_PER_CHUNK, 512), jnp.float32),
            pltpu.VMEM((ROWS_PER_CHUNK, 512), jnp.int32),
            pltpu.VMEM((4 * B // 128, 128), jnp.float32),
            pltpu.SemaphoreType.DMA,
            pltpu.SemaphoreType.DMA,
            pltpu.SemaphoreType.DMA,
            pltpu.SemaphoreType.DMA,
            pltpu.SemaphoreType.DMA,
            pltpu.SemaphoreType.DMA,
        ],
    )


def _tc_post_kernel(h_ref, o_ref):
    R = B // 128
    x = h_ref[...]
    t = jnp.sum(x, axis=0)
    cp = t[0:R]
    cn = t[R:2 * R]
    sp = t[2 * R:3 * R]
    sn = t[3 * R:4 * R]

    row = lax.broadcasted_iota(jnp.int32, (R, R), 0)
    col = lax.broadcasted_iota(jnp.int32, (R, R), 1)
    upper_inc = (row <= col).astype(jnp.float32)
    lower_str = (col < row).astype(jnp.float32)

    def prefix_inclusive(a):
        inc = jnp.dot(a, upper_inc, preferred_element_type=jnp.float32)
        rowtot = jnp.sum(a, axis=1, keepdims=True)
        off = jnp.dot(lower_str, rowtot, preferred_element_type=jnp.float32)
        return inc + off

    ipn = prefix_inclusive(cn)
    ipp = prefix_inclusive(cp)
    tn = jnp.sum(cn)
    tp = jnp.sum(cp)
    gn = tn - ipn
    gp = tp - ipp
    p_tot = tp

    pos_den = jnp.maximum(p_tot + gn + 0.5 * cn, 0.5)
    pos_term = sp / pos_den
    nn_hat = gn + 0.5 * (cn - 1.0)
    neg_den = jnp.maximum((p_tot + nn_hat) * (p_tot + nn_hat + 1.0), 0.5)
    neg_term = sn * (p_tot - (gp + 0.5 * cp)) / neg_den

    o_ref[0, 0] = jnp.sum(pos_term + neg_term)


def kernel(outputs, targets):
    labels = targets.astype(jnp.int32)
    hists = _sc_hist()(outputs, labels)
    loss = pl.pallas_call(
        _tc_post_kernel,
        out_shape=jax.ShapeDtypeStruct((1, 1), jnp.float32),
        out_specs=pl.BlockSpec(memory_space=pltpu.SMEM),
    )(hists)
    return loss.reshape(())


# speedup vs baseline: 1.0264x; 1.0264x over previous
import functools

import jax
import jax.numpy as jnp
from jax import lax
from jax.experimental import pallas as pl
from jax.experimental.pallas import tpu as pltpu
from jax.experimental.pallas import tpu_sc as plsc

N = 16 * 1 * 512 * 512
LBITS = 14
B = 1 << LBITS
NW = 32
ROWS_PER_CHUNK = 16
NCH = 256 // ROWS_PER_CHUNK
LANES = 16

_INT_MIN = -2147483648


def _sc_hist_kernel(logits_hbm, labels_hbm, out_hbm,
                    lbuf0, ybuf0, lbuf1, ybuf1, hist,
                    sl0, sy0, sl1, sy1):
    nc = 2
    wid = lax.axis_index("s") * nc + lax.axis_index("c")
    img = wid // 2
    row_base = (wid % 2) * 256

    zeros16 = jnp.zeros((LANES,), jnp.float32)
    ones16 = jnp.full((LANES,), 1.0, jnp.float32)
    sh31 = jnp.full((LANES,), 31, jnp.int32)
    sh_binrow = jnp.full((LANES,), 32 - LBITS + 7, jnp.int32)
    sh_bin = jnp.full((LANES,), 32 - LBITS, jnp.int32)
    sh7 = jnp.full((LANES,), 7, jnp.int32)

    @plsc.parallel_loop(0, 4 * B // 128, unroll=8)
    def _(r):
        for j in range(128 // LANES):
            hist[r, pl.ds(j * LANES, LANES)] = zeros16

    def process(lb, yb):
        @plsc.parallel_loop(0, ROWS_PER_CHUNK * (512 // LANES), unroll=8)
        def _(i):
            r = i // (512 // LANES)
            c = (i % (512 // LANES)) * LANES
            x = lb[r, pl.ds(c, LANES)]
            y = yb[r, pl.ds(c, LANES)]
            y1 = y ^ 1
            xb = lax.bitcast_convert_type(x, jnp.int32)
            e = 1.0 - lax.bitcast_convert_type(
                xb ^ lax.shift_left(y1, sh31), jnp.float32)
            relu_e = jnp.maximum(e, 0.0)
            b = lax.bitcast_convert_type(e, jnp.int32)
            keyu = b ^ (lax.shift_right_arithmetic(b, sh31)
                        | jnp.int32(_INT_MIN))
            col = lax.shift_right_logical(keyu, sh_bin) & 127
            row_cnt = (lax.shift_right_logical(keyu, sh_binrow)
                       | lax.shift_left(y1, sh7))
            row_sum = row_cnt | (2 * B // 128)
            plsc.addupdate_scatter(hist, [row_cnt, col], ones16)
            plsc.addupdate_scatter(hist, [row_sum, col], relu_e)

    def start(g, lb, yb, sl, sy):
        r0 = row_base + g * ROWS_PER_CHUNK
        pltpu.async_copy(
            logits_hbm.at[img, 0, pl.ds(r0, ROWS_PER_CHUNK), :], lb, sl)
        pltpu.async_copy(
            labels_hbm.at[img, 0, pl.ds(r0, ROWS_PER_CHUNK), :], yb, sy)

    def wait(lb, yb, sl, sy):
        src_l = logits_hbm.at[img, 0, pl.ds(row_base, ROWS_PER_CHUNK), :]
        src_y = labels_hbm.at[img, 0, pl.ds(row_base, ROWS_PER_CHUNK), :]
        pltpu.make_async_copy(src_l, lb, sl).wait()
        pltpu.make_async_copy(src_y, yb, sy).wait()

    start(0, lbuf0, ybuf0, sl0, sy0)

    def pair_body(h, c):
        g0 = 2 * h
        wait(lbuf0, ybuf0, sl0, sy0)
        start(g0 + 1, lbuf1, ybuf1, sl1, sy1)
        process(lbuf0, ybuf0)
        wait(lbuf1, ybuf1, sl1, sy1)
        start(jnp.minimum(g0 + 2, NCH - 1), lbuf0, ybuf0, sl0, sy0)
        process(lbuf1, ybuf1)
        return c

    lax.fori_loop(0, NCH // 2, pair_body, 0)
    wait(lbuf0, ybuf0, sl0, sy0)
    pltpu.sync_copy(hist, out_hbm.at[wid])


@functools.cache
def _sc_hist():
    return pl.kernel(
        _sc_hist_kernel,
        mesh=plsc.VectorSubcoreMesh(core_axis_name="c", subcore_axis_name="s"),
        compiler_params=pltpu.CompilerParams(needs_layout_passes=False),
        out_type=jax.ShapeDtypeStruct((NW, 4 * B // 128, 128), jnp.float32),
        scratch_types=[
            pltpu.VMEM((ROWS_PER_CHUNK, 512), jnp.float32),
            pltpu.VMEM((ROWS_PER_CHUNK, 512), jnp.int32),
            pltpu.VMEM((ROWS_PER_CHUNK, 512), jnp.float32),
            pltpu.VMEM((ROWS_PER_CHUNK, 512), jnp.int32),
            pltpu.VMEM((4 * B // 128, 128), jnp.float32),
            pltpu.SemaphoreType.DMA,
            pltpu.SemaphoreType.DMA,
            pltpu.SemaphoreType.DMA,
            pltpu.SemaphoreType.DMA,
        ],
    )


def _tc_post_kernel(h_ref, o_ref):
    R = B // 128
    x = h_ref[...]
    t = jnp.sum(x, axis=0)
    cp = t[0:R]
    cn = t[R:2 * R]
    sp = t[2 * R:3 * R]
    sn = t[3 * R:4 * R]

    row = lax.broadcasted_iota(jnp.int32, (R, R), 0)
    col = lax.broadcasted_iota(jnp.int32, (R, R), 1)
    upper_inc = (row <= col).astype(jnp.float32)
    lower_str = (col < row).astype(jnp.float32)

    def prefix_inclusive(a):
        inc = jnp.dot(a, upper_inc, preferred_element_type=jnp.float32)
        rowtot = jnp.sum(a, axis=1, keepdims=True)
        off = jnp.dot(lower_str, rowtot, preferred_element_type=jnp.float32)
        return inc + off

    ipn = prefix_inclusive(cn)
    ipp = prefix_inclusive(cp)
    tn = jnp.sum(cn)
    tp = jnp.sum(cp)
    gn = tn - ipn
    gp = tp - ipp
    p_tot = tp

    pos_den = jnp.maximum(p_tot + gn + 0.5 * cn, 0.5)
    pos_term = sp / pos_den
    nn_hat = gn + 0.5 * (cn - 1.0)
    neg_den = jnp.maximum((p_tot + nn_hat) * (p_tot + nn_hat + 1.0), 0.5)
    neg_term = sn * (p_tot - (gp + 0.5 * cp)) / neg_den

    o_ref[0, 0] = jnp.sum(pos_term + neg_term)


def kernel(outputs, targets):
    labels = targets.astype(jnp.int32)
    hists = _sc_hist()(outputs, labels)
    loss = pl.pallas_call(
        _tc_post_kernel,
        out_shape=jax.ShapeDtypeStruct((1, 1), jnp.float32),
        out_specs=pl.BlockSpec(memory_space=pltpu.SMEM),
    )(hists)
    return loss.reshape(())
